# baseline (speedup 1.0000x reference)
import functools

import jax
import jax.numpy as jnp
from jax import lax
from jax.experimental import pallas as pl
from jax.experimental.pallas import tpu as pltpu
from jax.experimental.pallas import tpu_sc as plsc

N = 100000
NUM_SEG = 1024
NSLAB = 5
GRID = 5
BLKR = 4000
CHUNK = GRID * BLKR
EPAD = 4096
EROW = GRID * EPAD
UNIT = 2000
G_UNIT = UNIT // 16
NUNITS = N // UNIT
SEG_PER_W = NUM_SEG // 16


def _mlp_body(*refs):
    xc = refs[:NSLAB]
    wa_ref, b1_ref, w2t_ref, b2_ref, o_ref = refs[NSLAB:]
    h = jnp.dot(xc[0][...], wa_ref[pl.ds(0, 64), :],
                preferred_element_type=jnp.float32)
    for c in range(1, NSLAB):
        h = h + jnp.dot(xc[c][...], wa_ref[pl.ds(c * 64, 64), :],
                        preferred_element_type=jnp.float32)
    h = h + b1_ref[...]
    sil = h * (1.0 / (1.0 + jnp.exp(-h)))
    e8t = lax.dot_general(w2t_ref[...], sil, (((1,), (1,)), ((), ())),
                          preferred_element_type=jnp.float32)
    o_ref[:, pl.ds(0, BLKR)] = e8t + b2_ref[...]


def _make_xspec(c):
    return pl.BlockSpec((BLKR, 64), lambda i, c=c: (GRID * c + i, 0))


def _mlp(x, wa, b1t, w2t, b2t):
    return pl.pallas_call(
        _mlp_body,
        grid=(GRID,),
        in_specs=[_make_xspec(c) for c in range(NSLAB)] + [
            pl.BlockSpec((64 * NSLAB, 128), lambda i: (0, 0)),
            pl.BlockSpec((1, 128), lambda i: (0, 0)),
            pl.BlockSpec((8, 128), lambda i: (0, 0)),
            pl.BlockSpec((1, 1), lambda i: (0, 0)),
        ],
        out_specs=pl.BlockSpec((8, EPAD), lambda i: (0, i)),
        out_shape=jax.ShapeDtypeStruct((8, EROW), jnp.float32),
    )(x, x, x, x, x, wa, b1t, w2t, b2t)


def _seg_body(e_hbm, data_hbm, out_hbm, ev, idv, bins, partial, red,
              seg_out, shared, sem):
    sid = lax.axis_index("s")
    u0 = jnp.where(sid < 2, 4 * sid, 8 + 3 * (sid - 2))
    nu = jnp.where(sid < 2, 4, 3)

    lanes = lax.iota(jnp.int32, 16)
    zero16 = jnp.zeros((16,), jnp.float32)

    def _z(j, _):
        for r in range(16):
            bins[r, pl.ds(j * 16, 16)] = zero16
        return 0
    lax.fori_loop(0, NUM_SEG // 16, _z, 0)

    def _unit(k, _):
        u = u0 + k
        cell = u // 2
        h = u % 2
        c = cell // GRID
        i = cell % GRID
        ebase = i * EPAD + h * UNIT
        nbase = c * CHUNK + i * BLKR + h * UNIT
        cp_e = pltpu.make_async_copy(e_hbm.at[c, pl.ds(ebase, UNIT)], ev, sem)
        cp_i = pltpu.make_async_copy(data_hbm.at[pl.ds(nbase, UNIT)], idv, sem)
        cp_e.start()
        cp_i.start()
        cp_e.wait()
        cp_i.wait()

        def _group(g, _):
            row0 = g * 16
            e = ev[pl.ds(row0, 16)]
            ids = idv[pl.ds(row0, 16)]
            plsc.addupdate_scatter(bins, [lanes, ids], e)
            return 0

        lax.fori_loop(0, G_UNIT, _group, 0)
        return 0

    lax.fori_loop(0, nu, _unit, 0)

    def _red(gj, _):
        c0 = gj * 16
        acc = bins[0, pl.ds(c0, 16)]
        for r in range(1, 16):
            acc = acc + bins[r, pl.ds(c0, 16)]
        partial[pl.ds(c0, 16)] = acc
        return 0
    lax.fori_loop(0, NUM_SEG // 16, _red, 0)

    pltpu.sync_copy(partial, shared.at[sid])
    plsc.subcore_barrier()
    c0 = sid * SEG_PER_W
    pltpu.sync_copy(shared.at[:, pl.ds(c0, SEG_PER_W)], red)
    for j in range(SEG_PER_W // 16):
        acc = red[0, pl.ds(j * 16, 16)]
        for r in range(1, 16):
            acc = acc + red[r, pl.ds(j * 16, 16)]
        seg_out[pl.ds(j * 16, 16)] = acc
    pltpu.sync_copy(seg_out, out_hbm.at[pl.ds(c0, SEG_PER_W)])


@functools.partial(
    pl.kernel,
    mesh=plsc.VectorSubcoreMesh(core_axis_name="c", subcore_axis_name="s",
                                num_cores=1),
    out_type=jax.ShapeDtypeStruct((NUM_SEG,), jnp.float32),
    scratch_types=[
        pltpu.VMEM((UNIT,), jnp.float32),
        pltpu.VMEM((UNIT,), jnp.int32),
        pltpu.VMEM((16, NUM_SEG), jnp.float32),
        pltpu.VMEM((NUM_SEG,), jnp.float32),
        pltpu.VMEM((16, SEG_PER_W), jnp.float32),
        pltpu.VMEM((SEG_PER_W,), jnp.float32),
        pltpu.VMEM_SHARED((16, NUM_SEG), jnp.float32),
        pltpu.SemaphoreType.DMA,
    ],
    compiler_params=pltpu.CompilerParams(use_tc_tiling_on_sc=False,
                                         needs_layout_passes=False),
)
def _sc_segsum(e_hbm, data_hbm, out_hbm, ev, idv, bins, partial, red,
               seg_out, shared, sem):
    _seg_body(e_hbm, data_hbm, out_hbm, ev, idv, bins, partial, red,
              seg_out, shared, sem)


def kernel(x, data, pos, W1, b1, W2, b2):
    data_i = data.astype(jnp.int32)
    w1blk = jnp.zeros((64, 16), jnp.float32).at[48:64, :].set(
        W1.T.astype(jnp.float32))
    eye = jnp.eye(NSLAB, dtype=jnp.float32)
    wa = jnp.zeros((64 * NSLAB, 128), jnp.float32).at[:, :16 * NSLAB].set(
        jnp.kron(eye, w1blk))
    b1t = jnp.zeros((1, 128), jnp.float32).at[0, :16 * NSLAB].set(
        jnp.tile(b1.astype(jnp.float32), NSLAB))
    w2t = jnp.zeros((8, 128), jnp.float32).at[:NSLAB, :16 * NSLAB].set(
        jnp.kron(eye, W2.astype(jnp.float32).reshape(1, 16)))
    b2t = b2.astype(jnp.float32).reshape(1, 1)

    e8t = _mlp(x, wa, b1t, w2t, b2t)
    E = _sc_segsum(e8t, data_i)
    F = jnp.zeros((N, 3), jnp.float32)
    return (E.reshape(NUM_SEG, 1), F)


# speedup vs baseline: 2.3836x; 2.3836x over previous
import functools

import jax
import jax.numpy as jnp
from jax import lax
from jax.experimental import pallas as pl
from jax.experimental.pallas import tpu as pltpu
from jax.experimental.pallas import tpu_sc as plsc

N = 100000
NUM_SEG = 1024
UNIT = 2000
G_UNIT = UNIT // 16
SEG_PER_W = NUM_SEG // 16


def _mlp_body(st_ref, w1_ref, b1_ref, w2_ref, b2_ref, o_ref):
    ht = lax.dot_general(w1_ref[...], st_ref[...], (((1,), (0,)), ((), ())),
                         preferred_element_type=jnp.float32) + b1_ref[...]
    sil = ht * (1.0 / (1.0 + jnp.exp(-ht)))
    et = lax.dot_general(w2_ref[...], sil, (((1,), (0,)), ((), ())),
                         preferred_element_type=jnp.float32)
    o_ref[...] = et + b2_ref[...]


def _mlp(xt, w1, b1c, w2r, b2t):
    return pl.pallas_call(
        _mlp_body,
        grid=(1,),
        in_specs=[
            pl.BlockSpec((16, N), lambda i: (3, 0)),
            pl.BlockSpec((16, 16), lambda i: (0, 0)),
            pl.BlockSpec((16, 1), lambda i: (0, 0)),
            pl.BlockSpec((1, 16), lambda i: (0, 0)),
            pl.BlockSpec((1, 1), lambda i: (0, 0)),
        ],
        out_specs=pl.BlockSpec((1, N), lambda i: (0, 0)),
        out_shape=jax.ShapeDtypeStruct((1, N), jnp.float32),
    )(xt, w1, b1c, w2r, b2t)


def _seg_body(e_hbm, data_hbm, out_hbm, ev, idv, bins, partial, red,
              seg_out, shared, sem):
    sid = lax.axis_index("s")
    u0 = jnp.where(sid < 2, 4 * sid, 8 + 3 * (sid - 2))
    nu = jnp.where(sid < 2, 4, 3)

    lanes = lax.iota(jnp.int32, 16)
    zero16 = jnp.zeros((16,), jnp.float32)

    def _z(j, _):
        for r in range(16):
            bins[r, pl.ds(j * 16, 16)] = zero16
        return 0
    lax.fori_loop(0, NUM_SEG // 16, _z, 0)

    def _unit(k, _):
        base = (u0 + k) * UNIT
        cp_e = pltpu.make_async_copy(e_hbm.at[0, pl.ds(base, UNIT)], ev, sem)
        cp_i = pltpu.make_async_copy(data_hbm.at[pl.ds(base, UNIT)], idv, sem)
        cp_e.start()
        cp_i.start()
        cp_e.wait()
        cp_i.wait()

        def _group(g, _):
            row0 = g * 16
            e = ev[pl.ds(row0, 16)]
            ids = idv[pl.ds(row0, 16)]
            plsc.addupdate_scatter(bins, [lanes, ids], e)
            return 0

        lax.fori_loop(0, G_UNIT, _group, 0)
        return 0

    lax.fori_loop(0, nu, _unit, 0)

    def _red(gj, _):
        c0 = gj * 16
        acc = bins[0, pl.ds(c0, 16)]
        for r in range(1, 16):
            acc = acc + bins[r, pl.ds(c0, 16)]
        partial[pl.ds(c0, 16)] = acc
        return 0
    lax.fori_loop(0, NUM_SEG // 16, _red, 0)

    pltpu.sync_copy(partial, shared.at[sid])
    plsc.subcore_barrier()
    c0 = sid * SEG_PER_W
    pltpu.sync_copy(shared.at[:, pl.ds(c0, SEG_PER_W)], red)
    for j in range(SEG_PER_W // 16):
        acc = red[0, pl.ds(j * 16, 16)]
        for r in range(1, 16):
            acc = acc + red[r, pl.ds(j * 16, 16)]
        seg_out[pl.ds(j * 16, 16)] = acc
    pltpu.sync_copy(seg_out, out_hbm.at[pl.ds(c0, SEG_PER_W)])


@functools.partial(
    pl.kernel,
    mesh=plsc.VectorSubcoreMesh(core_axis_name="c", subcore_axis_name="s",
                                num_cores=1),
    out_type=jax.ShapeDtypeStruct((NUM_SEG,), jnp.float32),
    scratch_types=[
        pltpu.VMEM((UNIT,), jnp.float32),
        pltpu.VMEM((UNIT,), jnp.int32),
        pltpu.VMEM((16, NUM_SEG), jnp.float32),
        pltpu.VMEM((NUM_SEG,), jnp.float32),
        pltpu.VMEM((16, SEG_PER_W), jnp.float32),
        pltpu.VMEM((SEG_PER_W,), jnp.float32),
        pltpu.VMEM_SHARED((16, NUM_SEG), jnp.float32),
        pltpu.SemaphoreType.DMA,
    ],
    compiler_params=pltpu.CompilerParams(use_tc_tiling_on_sc=False,
                                         needs_layout_passes=False),
)
def _sc_segsum(e_hbm, data_hbm, out_hbm, ev, idv, bins, partial, red,
               seg_out, shared, sem):
    _seg_body(e_hbm, data_hbm, out_hbm, ev, idv, bins, partial, red,
              seg_out, shared, sem)


def kernel(x, data, pos, W1, b1, W2, b2):
    data_i = data.astype(jnp.int32)
    xt = x.T
    w1 = W1.astype(jnp.float32)
    b1c = b1.astype(jnp.float32).reshape(16, 1)
    w2r = W2.astype(jnp.float32).reshape(1, 16)
    b2t = b2.astype(jnp.float32).reshape(1, 1)

    et = _mlp(xt, w1, b1c, w2r, b2t)
    E = _sc_segsum(et, data_i)
    F = jnp.zeros((N, 3), jnp.float32)
    return (E.reshape(NUM_SEG, 1), F)


# speedup vs baseline: 2.5666x; 1.0768x over previous
import functools

import jax
import jax.numpy as jnp
from jax import lax
from jax.experimental import pallas as pl
from jax.experimental.pallas import tpu as pltpu
from jax.experimental.pallas import tpu_sc as plsc

N = 100000
NUM_SEG = 1024
UNIT = 2000
G_UNIT = UNIT // 16
NUNITS = N // UNIT
SEG_PER_W = NUM_SEG // 16


def _mlp_body(st_ref, w1_ref, b1_ref, w2_ref, b2_ref, o_ref):
    ht = lax.dot_general(w1_ref[...], st_ref[...], (((1,), (0,)), ((), ())),
                         preferred_element_type=jnp.float32) + b1_ref[...]
    sil = ht * (1.0 / (1.0 + jnp.exp(-ht)))
    et = lax.dot_general(w2_ref[...], sil, (((1,), (0,)), ((), ())),
                         preferred_element_type=jnp.float32)
    o_ref[...] = et + b2_ref[...]


def _mlp(xt, w1, b1c, w2r, b2t):
    return pl.pallas_call(
        _mlp_body,
        grid=(1,),
        in_specs=[
            pl.BlockSpec((16, N), lambda i: (3, 0)),
            pl.BlockSpec((16, 16), lambda i: (0, 0)),
            pl.BlockSpec((16, 1), lambda i: (0, 0)),
            pl.BlockSpec((1, 16), lambda i: (0, 0)),
            pl.BlockSpec((1, 1), lambda i: (0, 0)),
        ],
        out_specs=pl.BlockSpec((1, N), lambda i: (0, 0)),
        out_shape=jax.ShapeDtypeStruct((1, N), jnp.float32),
    )(xt, w1, b1c, w2r, b2t)


def _seg_body(e_hbm, data_hbm, out_hbm, ev0, ev1, idv0, idv1, bins, partial,
              red, seg_out, shared, sem0, sem1):
    sid = lax.axis_index("s")
    u0 = jnp.where(sid < 2, 4 * sid, 8 + 3 * (sid - 2))
    nu = jnp.where(sid < 2, 4, 3)

    evs = (ev0, ev1)
    idvs = (idv0, idv1)
    sems = (sem0, sem1)

    def _cp(k):
        base = jnp.minimum(u0 + k, NUNITS - 1) * UNIT
        return (pltpu.make_async_copy(e_hbm.at[0, pl.ds(base, UNIT)],
                                      evs[k % 2], sems[k % 2]),
                pltpu.make_async_copy(data_hbm.at[pl.ds(base, UNIT)],
                                      idvs[k % 2], sems[k % 2]))

    cps = [_cp(k) for k in range(4)]
    cps[0][0].start()
    cps[0][1].start()
    cps[1][0].start()
    cps[1][1].start()

    lanes = lax.iota(jnp.int32, 16)
    zero16 = jnp.zeros((16,), jnp.float32)

    def _z(j, _):
        for r in range(16):
            bins[r, pl.ds(j * 16, 16)] = zero16
        return 0
    lax.fori_loop(0, NUM_SEG // 16, _z, 0)

    for k in range(4):
        @pl.when(jnp.asarray(k) < nu)
        def _(k=k):
            cps[k][0].wait()
            cps[k][1].wait()
            ev = evs[k % 2]
            idv = idvs[k % 2]

            def _group(g, _):
                row0 = g * 16
                e = ev[pl.ds(row0, 16)]
                ids = idv[pl.ds(row0, 16)]
                plsc.addupdate_scatter(bins, [lanes, ids], e)
                return 0

            lax.fori_loop(0, G_UNIT, _group, 0)

            if k + 2 < 4:

                @pl.when(jnp.asarray(k + 2) < nu)
                def _():
                    cps[k + 2][0].start()
                    cps[k + 2][1].start()

    def _red(gj, _):
        c0 = gj * 16
        acc = bins[0, pl.ds(c0, 16)]
        for r in range(1, 16):
            acc = acc + bins[r, pl.ds(c0, 16)]
        partial[pl.ds(c0, 16)] = acc
        return 0
    lax.fori_loop(0, NUM_SEG // 16, _red, 0)

    pltpu.sync_copy(partial, shared.at[sid])
    plsc.subcore_barrier()
    c0 = sid * SEG_PER_W
    pltpu.sync_copy(shared.at[:, pl.ds(c0, SEG_PER_W)], red)
    for j in range(SEG_PER_W // 16):
        acc = red[0, pl.ds(j * 16, 16)]
        for r in range(1, 16):
            acc = acc + red[r, pl.ds(j * 16, 16)]
        seg_out[pl.ds(j * 16, 16)] = acc
    pltpu.sync_copy(seg_out, out_hbm.at[pl.ds(c0, SEG_PER_W)])


@functools.partial(
    pl.kernel,
    mesh=plsc.VectorSubcoreMesh(core_axis_name="c", subcore_axis_name="s",
                                num_cores=1),
    out_type=jax.ShapeDtypeStruct((NUM_SEG,), jnp.float32),
    scratch_types=[
        pltpu.VMEM((UNIT,), jnp.float32),
        pltpu.VMEM((UNIT,), jnp.float32),
        pltpu.VMEM((UNIT,), jnp.int32),
        pltpu.VMEM((UNIT,), jnp.int32),
        pltpu.VMEM((16, NUM_SEG), jnp.float32),
        pltpu.VMEM((NUM_SEG,), jnp.float32),
        pltpu.VMEM((16, SEG_PER_W), jnp.float32),
        pltpu.VMEM((SEG_PER_W,), jnp.float32),
        pltpu.VMEM_SHARED((16, NUM_SEG), jnp.float32),
        pltpu.SemaphoreType.DMA,
        pltpu.SemaphoreType.DMA,
    ],
    compiler_params=pltpu.CompilerParams(use_tc_tiling_on_sc=False,
                                         needs_layout_passes=False),
)
def _sc_segsum(e_hbm, data_hbm, out_hbm, ev0, ev1, idv0, idv1, bins, partial,
               red, seg_out, shared, sem0, sem1):
    _seg_body(e_hbm, data_hbm, out_hbm, ev0, ev1, idv0, idv1, bins, partial,
              red, seg_out, shared, sem0, sem1)


def kernel(x, data, pos, W1, b1, W2, b2):
    data_i = data.astype(jnp.int32)
    xt = x.T
    w1 = W1.astype(jnp.float32)
    b1c = b1.astype(jnp.float32).reshape(16, 1)
    w2r = W2.astype(jnp.float32).reshape(1, 16)
    b2t = b2.astype(jnp.float32).reshape(1, 1)

    et = _mlp(xt, w1, b1c, w2r, b2t)
    E = _sc_segsum(et, data_i)
    F = jnp.zeros((N, 3), jnp.float32)
    return (E.reshape(NUM_SEG, 1), F)
